# baseline (speedup 1.0000x reference)
import functools

import jax
import jax.numpy as jnp
from jax import lax
from jax.experimental import pallas as pl
from jax.experimental.pallas import tpu as pltpu
from jax.experimental.pallas import tpu_sc as plsc

_VOCAB = 1000000
_D = 64
_DP = 64
_B = 1024
_N = 200

_NC = 2
_NS = 16
_NW = _NC * _NS
_SEQ_PER_W = _B // _NW
_LANES = 16
_NBUF = 2


def _positional_encoding(n, d):
    pos = jnp.arange(n, dtype=jnp.float32)
    two_i = 2.0 * jnp.floor(jnp.arange(d, dtype=jnp.float32) / 2.0)
    angles = pos[:, None] / jnp.power(10000.0, two_i / float(d))
    even = (jnp.arange(d) % 2) == 0
    return jnp.where(even[None, :], jnp.sin(angles), jnp.cos(angles))


_mesh = plsc.VectorSubcoreMesh(core_axis_name="c", subcore_axis_name="s")


@functools.partial(
    pl.kernel,
    mesh=_mesh,
    out_type=jax.ShapeDtypeStruct((_B * _N, _D), jnp.float32),
    scratch_types=[
        pltpu.VMEM((_SEQ_PER_W * _N,), jnp.int32),
        pltpu.VMEM((_N, _D), jnp.float32),
        [pltpu.VMEM((_N, _DP), jnp.float32)] * _NBUF,
        [pltpu.VMEM((_N, _D), jnp.float32)] * _NBUF,
        [pltpu.SemaphoreType.DMA] * _NBUF,
        [pltpu.SemaphoreType.DMA] * _NBUF,
    ],
    compiler_params=pltpu.CompilerParams(use_tc_tiling_on_sc=False),
)
def _embed(idx_hbm, table_hbm, enc_hbm, out_hbm, idx_v, enc_v, rows, outs, gsem, osem):
    wid = lax.axis_index("s") * _NC + lax.axis_index("c")
    base = wid * _SEQ_PER_W * _N
    pltpu.sync_copy(enc_hbm, enc_v)
    pltpu.sync_copy(idx_hbm.at[pl.ds(base, _SEQ_PER_W * _N)], idx_v)

    def start_gather(s, b):
        return pltpu.async_copy(
            table_hbm.at[idx_v.at[pl.ds(s * _N, _N)]], rows[b], gsem[b])

    def wait_gather(s, b):
        pltpu.make_async_copy(
            table_hbm.at[idx_v.at[pl.ds(s * _N, _N)]], rows[b], gsem[b]).wait()

    def start_out(s, b):
        return pltpu.async_copy(
            outs[b], out_hbm.at[pl.ds(base + s * _N, _N)], osem[b])

    def wait_out(s, b):
        pltpu.make_async_copy(
            outs[b], out_hbm.at[pl.ds(base + s * _N, _N)], osem[b]).wait()

    start_gather(0, 0)
    start_gather(1, 1)

    def iter_body(i, carry):
        for b in range(_NBUF):
            s = _NBUF * i + b
            wait_gather(s, b)

            @pl.when(i > 0)
            def _():
                wait_out(s - _NBUF, b)

            @plsc.parallel_loop(0, _N, 1, unroll=8)
            def _(r):
                for c in range(_D // _LANES):
                    sl = pl.ds(c * _LANES, _LANES)
                    outs[b][r, sl] = rows[b][r, sl] + enc_v[r, sl]

            @pl.when(i < _SEQ_PER_W // _NBUF - 1)
            def _():
                start_gather(s + _NBUF, b)

            start_out(s, b)
        return carry

    lax.fori_loop(0, _SEQ_PER_W // _NBUF, iter_body, 0)

    for b in range(_NBUF):
        wait_out(_SEQ_PER_W - _NBUF + b, b)


_HALF = 500096
_VPAD = 2 * _HALF


def _tr_body(xl_ref, xr_ref, o_ref):
    o_ref[...] = jnp.concatenate([xl_ref[...].T, xr_ref[...].T], axis=1)


_transpose_tc = pl.pallas_call(
    _tr_body,
    grid=(3907,),
    in_specs=[
        pl.BlockSpec((_D, 2 * _D), lambda k: (0, k)),
        pl.BlockSpec((_D, 2 * _D), lambda k: (0, jnp.minimum(k + 3907, 7812))),
    ],
    out_specs=pl.BlockSpec((2 * _D, 2 * _D), lambda k: (k, 0)),
    out_shape=jax.ShapeDtypeStruct((_HALF, 2 * _D), jnp.float32),
)


def kernel(x, W):
    enc = _positional_encoding(_N, _D)
    Wc = _transpose_tc(W.T, W.T)
    table = Wc.reshape(_VPAD, _D)
    xf = x.reshape(_B * _N)
    idx2 = jnp.where(xf < _HALF, 2 * xf, 2 * xf - (_VPAD - 1))
    out = _embed(idx2, table, enc)
    return out.reshape(_B, _N, _D)


# speedup vs baseline: 3.2746x; 3.2746x over previous
import functools

import jax
import jax.numpy as jnp
from jax import lax
from jax.experimental import pallas as pl
from jax.experimental.pallas import tpu as pltpu
from jax.experimental.pallas import tpu_sc as plsc

_VOCAB = 1000000
_D = 64
_DP = 128
_B = 1024
_N = 200

_NC = 2
_NS = 16
_NW = _NC * _NS
_SEQ_PER_W = _B // _NW
_LANES = 16
_NBUF = 2


def _positional_encoding(n, d):
    pos = jnp.arange(n, dtype=jnp.float32)
    two_i = 2.0 * jnp.floor(jnp.arange(d, dtype=jnp.float32) / 2.0)
    angles = pos[:, None] / jnp.power(10000.0, two_i / float(d))
    even = (jnp.arange(d) % 2) == 0
    return jnp.where(even[None, :], jnp.sin(angles), jnp.cos(angles))


_mesh = plsc.VectorSubcoreMesh(core_axis_name="c", subcore_axis_name="s")


@functools.partial(
    pl.kernel,
    mesh=_mesh,
    out_type=jax.ShapeDtypeStruct((_B * _N, _D), jnp.float32),
    scratch_types=[
        pltpu.VMEM((_SEQ_PER_W * _N,), jnp.int32),
        pltpu.VMEM((_N, _D), jnp.float32),
        [pltpu.VMEM((_N, _DP), jnp.float32)] * _NBUF,
        [pltpu.VMEM((_N, _D), jnp.float32)] * _NBUF,
        [pltpu.SemaphoreType.DMA] * _NBUF,
        [pltpu.SemaphoreType.DMA] * _NBUF,
    ],
    compiler_params=pltpu.CompilerParams(use_tc_tiling_on_sc=False),
)
def _embed(idx_hbm, table_hbm, enc_hbm, out_hbm, idx_v, enc_v, rows, outs, gsem, osem):
    wid = lax.axis_index("s") * _NC + lax.axis_index("c")
    base = wid * _SEQ_PER_W * _N
    pltpu.sync_copy(enc_hbm, enc_v)
    pltpu.sync_copy(idx_hbm.at[pl.ds(base, _SEQ_PER_W * _N)], idx_v)

    def start_gather(s, b):
        return pltpu.async_copy(
            table_hbm.at[idx_v.at[pl.ds(s * _N, _N)]], rows[b], gsem[b])

    def wait_gather(s, b):
        pltpu.make_async_copy(
            table_hbm.at[idx_v.at[pl.ds(s * _N, _N)]], rows[b], gsem[b]).wait()

    def start_out(s, b):
        return pltpu.async_copy(
            outs[b], out_hbm.at[pl.ds(base + s * _N, _N)], osem[b])

    def wait_out(s, b):
        pltpu.make_async_copy(
            outs[b], out_hbm.at[pl.ds(base + s * _N, _N)], osem[b]).wait()

    start_gather(0, 0)
    start_gather(1, 1)

    def iter_body(i, carry):
        for b in range(_NBUF):
            s = _NBUF * i + b
            wait_gather(s, b)

            @pl.when(i > 0)
            def _():
                wait_out(s - _NBUF, b)

            @plsc.parallel_loop(0, _N, 1, unroll=8)
            def _(r):
                for c in range(_D // _LANES):
                    sl = pl.ds(c * _LANES, _LANES)
                    outs[b][r, sl] = rows[b][r, sl] + enc_v[r, sl]

            @pl.when(i < _SEQ_PER_W // _NBUF - 1)
            def _():
                start_gather(s + _NBUF, b)

            start_out(s, b)
        return carry

    lax.fori_loop(0, _SEQ_PER_W // _NBUF, iter_body, 0)

    for b in range(_NBUF):
        wait_out(_SEQ_PER_W - _NBUF + b, b)


def kernel(x, W):
    enc = _positional_encoding(_N, _D)
    Wp = jnp.pad(W, ((0, 0), (0, _DP - _D)))
    out = _embed(x.reshape(_B * _N), Wp, enc)
    return out.reshape(_B, _N, _D)


# speedup vs baseline: 3.5759x; 1.0920x over previous
import functools

import jax
import jax.numpy as jnp
from jax import lax
from jax.experimental import pallas as pl
from jax.experimental.pallas import tpu as pltpu
from jax.experimental.pallas import tpu_sc as plsc

_VOCAB = 1000000
_D = 64
_DP = 128
_B = 1024
_N = 200

_NC = 2
_NS = 16
_NW = _NC * _NS
_SEQ_PER_W = _B // _NW
_LANES = 16
_NBUF = 2


def _positional_encoding(n, d):
    pos = jnp.arange(n, dtype=jnp.float32)
    two_i = 2.0 * jnp.floor(jnp.arange(d, dtype=jnp.float32) / 2.0)
    angles = pos[:, None] / jnp.power(10000.0, two_i / float(d))
    even = (jnp.arange(d) % 2) == 0
    return jnp.where(even[None, :], jnp.sin(angles), jnp.cos(angles))


_mesh = plsc.VectorSubcoreMesh(core_axis_name="c", subcore_axis_name="s")


@functools.partial(
    pl.kernel,
    mesh=_mesh,
    out_type=jax.ShapeDtypeStruct((_B * _N, _DP), jnp.float32),
    scratch_types=[
        pltpu.VMEM((_SEQ_PER_W * _N,), jnp.int32),
        pltpu.VMEM((_N, _D), jnp.float32),
        [pltpu.VMEM((_N, _DP), jnp.float32)] * _NBUF,
        [pltpu.VMEM((_N, _DP), jnp.float32)] * _NBUF,
        [pltpu.SemaphoreType.DMA] * _NBUF,
        [pltpu.SemaphoreType.DMA] * _NBUF,
    ],
    compiler_params=pltpu.CompilerParams(use_tc_tiling_on_sc=False),
)
def _embed(idx_hbm, table_hbm, enc_hbm, out_hbm, idx_v, enc_v, rows, outs, gsem, osem):
    wid = lax.axis_index("s") * _NC + lax.axis_index("c")
    base = wid * _SEQ_PER_W * _N
    pltpu.sync_copy(enc_hbm, enc_v)
    pltpu.sync_copy(idx_hbm.at[pl.ds(base, _SEQ_PER_W * _N)], idx_v)

    def start_gather(s, b):
        return pltpu.async_copy(
            table_hbm.at[idx_v.at[pl.ds(s * _N, _N)]], rows[b], gsem[b])

    def wait_gather(s, b):
        pltpu.make_async_copy(
            table_hbm.at[idx_v.at[pl.ds(s * _N, _N)]], rows[b], gsem[b]).wait()

    def start_out(s, b):
        return pltpu.async_copy(
            outs[b], out_hbm.at[pl.ds(base + s * _N, _N)], osem[b])

    def wait_out(s, b):
        pltpu.make_async_copy(
            outs[b], out_hbm.at[pl.ds(base + s * _N, _N)], osem[b]).wait()

    start_gather(0, 0)
    start_gather(1, 1)

    def iter_body(i, carry):
        for b in range(_NBUF):
            s = _NBUF * i + b
            wait_gather(s, b)

            @pl.when(i > 0)
            def _():
                wait_out(s - _NBUF, b)

            @plsc.parallel_loop(0, _N, 1, unroll=8)
            def _(r):
                for c in range(_D // _LANES):
                    sl = pl.ds(c * _LANES, _LANES)
                    outs[b][r, sl] = rows[b][r, sl] + enc_v[r, sl]

            @pl.when(i < _SEQ_PER_W // _NBUF - 1)
            def _():
                start_gather(s + _NBUF, b)

            start_out(s, b)
        return carry

    lax.fori_loop(0, _SEQ_PER_W // _NBUF, iter_body, 0)

    for b in range(_NBUF):
        wait_out(_SEQ_PER_W - _NBUF + b, b)


def kernel(x, W):
    enc = _positional_encoding(_N, _D)
    Wp = jnp.pad(W, ((0, 0), (0, _DP - _D)))
    out = _embed(x.reshape(_B * _N), Wp, enc)
    return out.reshape(_B, _N, _DP)[:, :, :_D]
